# baseline (speedup 1.0000x reference)
import functools

import jax
import jax.numpy as jnp
from jax import lax
from jax.experimental import pallas as pl
from jax.experimental.pallas import tpu as pltpu
from jax.experimental.pallas import tpu_sc as plsc

N = 10000
E = 320000
F = 128
EF = 16

NC = 2
NS = 16
NW = NC * NS
C = 128
CA = 96
EPW = 10368
E_PAD = NW * EPW
ACC_ROWS = 10112
ZPT = ACC_ROWS // NS
RPT = 624
TAIL = N - NS * RPT
DW = 16


def _zero_vec():
    return jnp.zeros((16,), jnp.float32)


def _sc_degree(dst):
    mesh = plsc.VectorSubcoreMesh(core_axis_name="c", subcore_axis_name="s")

    @functools.partial(
        pl.kernel,
        out_type=jax.ShapeDtypeStruct((NC, N, F), jnp.float32),
        mesh=mesh,
        scratch_types=[
            pltpu.VMEM((2, C), jnp.int32),
            pltpu.VMEM((C, F), jnp.float32),
            pltpu.VMEM_SHARED((ACC_ROWS, F), jnp.float32),
            pltpu.SemaphoreType.DMA,
            pltpu.SemaphoreType.DMA,
        ],
    )
    def kern(dst_hbm, out_deg, dstv, ones, accd, ss0, ss1):
        cid = lax.axis_index("c")
        sid = lax.axis_index("s")
        wid = sid * NC + cid
        ss = (ss0, ss1)
        zero = _zero_vec()

        def zrow(r, carry):
            for j in range(F // 16):
                ones[r, pl.ds(j * 16, 16)] = zero
            return carry
        lax.fori_loop(0, C, zrow, 0)
        for i in range((ZPT + C - 1) // C):
            nr = min(C, ZPT - i * C)
            r0 = sid * ZPT + i * C
            pltpu.sync_copy(ones.at[pl.ds(0, nr)], accd.at[pl.ds(r0, nr)])

        one = jnp.ones((16,), jnp.float32)

        def initrow(r, carry):
            for j in range(F // 16):
                ones[r, pl.ds(j * 16, 16)] = one
            return carry
        lax.fori_loop(0, C, initrow, 0)
        plsc.subcore_barrier()

        base = wid * EPW

        cpw = EPW // C

        def sched(k, p, first):
            if not first:
                pltpu.make_async_copy(ones, accd.at[dstv.at[p]], ss[p]).wait()
            pltpu.sync_copy(dst_hbm.at[pl.ds(base + k * C, C)], dstv.at[p])
            pltpu.async_copy(ones, accd.at[dstv.at[p]], ss[p], add=True)

        sched(0, 0, True)
        sched(1, 1, True)

        def outer(i, carry):
            sched(2 * i + 2, 0, False)
            sched(2 * i + 3, 1, False)
            return carry
        lax.fori_loop(0, (cpw - 3) // 2, outer, 0)
        sched(cpw - 1, 0, False)
        pltpu.make_async_copy(ones, accd.at[dstv.at[0]], ss[0]).wait()
        pltpu.make_async_copy(ones, accd.at[dstv.at[1]], ss[1]).wait()
        plsc.subcore_barrier()

        def cpout(row, nr):
            pltpu.sync_copy(accd.at[pl.ds(row, nr)], ones.at[pl.ds(0, nr)])
            pltpu.sync_copy(ones.at[pl.ds(0, nr)], out_deg.at[cid, pl.ds(row, nr)])

        for i in range((RPT + C - 1) // C):
            cpout(sid * RPT + i * C, min(C, RPT - i * C))

        @pl.when(sid == NS - 1)
        def _tail():
            cpout(NS * RPT, TAIL)

    return kern(dst)


def _sc_edge_pass(table, src, dst, e=None):
    with_mul = e is not None
    cc = CA if with_mul else C
    cpw = EPW // cc
    mesh = plsc.VectorSubcoreMesh(core_axis_name="c", subcore_axis_name="s")

    scratch = [
        pltpu.VMEM((2, cc), jnp.int32),
        pltpu.VMEM((2, cc), jnp.int32),
        pltpu.VMEM((2, cc, F), jnp.float32),
    ]
    if with_mul:
        scratch.append(pltpu.VMEM((2, cc, F), jnp.float32))
    scratch.append(pltpu.VMEM_SHARED((ACC_ROWS, F), jnp.float32))
    nsem = 6 if with_mul else 4
    scratch += [pltpu.SemaphoreType.DMA] * nsem

    @functools.partial(
        pl.kernel,
        out_type=jax.ShapeDtypeStruct((NC, N, F), jnp.float32),
        mesh=mesh,
        scratch_types=scratch,
    )
    def kern(*args):
        if with_mul:
            (table_hbm, e_hbm, src_hbm, dst_hbm, out_sum,
             srcv, dstv, rows, ev, acc, g0, g1, s0, s1, e0, e1) = args
            sg, ss, se = (g0, g1), (s0, s1), (e0, e1)
        else:
            (table_hbm, src_hbm, dst_hbm, out_sum,
             srcv, dstv, rows, acc, g0, g1, s0, s1) = args
            sg, ss = (g0, g1), (s0, s1)
        cid = lax.axis_index("c")
        sid = lax.axis_index("s")
        wid = sid * NC + cid
        zero = _zero_vec()
        base = wid * EPW

        def initrow(r, carry):
            for j in range(F // 16):
                rows[0, r, pl.ds(j * 16, 16)] = zero
            return carry
        lax.fori_loop(0, cc, initrow, 0)

        for i in range((ZPT + cc - 1) // cc):
            nr = min(cc, ZPT - i * cc)
            r0 = sid * ZPT + i * cc
            pltpu.sync_copy(rows.at[0, pl.ds(0, nr)], acc.at[pl.ds(r0, nr)])
        plsc.subcore_barrier()

        def stage_a(k, p, first):
            if not first:
                pltpu.make_async_copy(rows.at[p], acc.at[dstv.at[p]],
                                      ss[p]).wait()
            off = base + k * cc
            pltpu.sync_copy(src_hbm.at[pl.ds(off, cc)], srcv.at[p])
            pltpu.sync_copy(dst_hbm.at[pl.ds(off, cc)], dstv.at[p])
            pltpu.async_copy(table_hbm.at[srcv.at[p]], rows.at[p], sg[p])
            if with_mul:
                pltpu.async_copy(e_hbm.at[pl.ds(off, cc)], ev.at[p], se[p])

        def stage_b(p):
            pltpu.make_async_copy(table_hbm.at[srcv.at[p]], rows.at[p],
                                  sg[p]).wait()
            if with_mul:
                pltpu.make_async_copy(e_hbm.at[pl.ds(base, cc)], ev.at[p],
                                      se[p]).wait()

                def mrow(r, c2):
                    for j in range(F // 16):
                        s = pl.ds(j * 16, 16)
                        rows[p, r, s] = rows[p, r, s] * ev[p, r, s]
                    return c2
                lax.fori_loop(0, cc, mrow, 0, unroll=2)
            pltpu.async_copy(rows.at[p], acc.at[dstv.at[p]], ss[p], add=True)

        stage_a(0, 0, True)
        stage_a(1, 1, True)

        nfull = (cpw - 2) // 2

        def outer(i, carry):
            k = 2 * i
            stage_b(0)
            stage_a(k + 2, 0, False)
            stage_b(1)
            stage_a(k + 3, 1, False)
            return carry
        lax.fori_loop(0, nfull, outer, 0)
        if cpw % 2 == 0:
            stage_b(0)
            stage_b(1)
        else:
            stage_b(0)
            stage_a(cpw - 1, 0, False)
            stage_b(1)
            stage_b(0)
        pltpu.make_async_copy(rows.at[0], acc.at[dstv.at[0]], ss[0]).wait()
        pltpu.make_async_copy(rows.at[1], acc.at[dstv.at[1]], ss[1]).wait()
        plsc.subcore_barrier()

        def cpout(row, nr):
            pltpu.sync_copy(acc.at[pl.ds(row, nr)], rows.at[0, pl.ds(0, nr)])
            pltpu.sync_copy(rows.at[0, pl.ds(0, nr)],
                            out_sum.at[cid, pl.ds(row, nr)])

        for i in range((RPT + cc - 1) // cc):
            cpout(sid * RPT + i * cc, min(cc, RPT - i * cc))

        @pl.when(sid == NS - 1)
        def _tail():
            cpout(NS * RPT, TAIL)

    if with_mul:
        return kern(table, e, src, dst)
    return kern(table, src, dst)


def _sc_edge_pass_a(nfeat, e, src, dst):
    return _sc_edge_pass(nfeat, src, dst, e=e)


def _sc_gather_scatter(table, src, dst):
    return _sc_edge_pass(table, src, dst)


def _tc_edge_encoder(efeat_pad, WeT, be2):
    BE = 2048
    grid = (E_PAD // BE,)

    def body(ef, w, b, o):
        o[...] = jax.nn.relu(
            jnp.dot(ef[...], w[...], preferred_element_type=jnp.float32) + b[...])

    return pl.pallas_call(
        body,
        grid=grid,
        in_specs=[
            pl.BlockSpec((BE, EF), lambda i: (i, 0)),
            pl.BlockSpec((EF, F), lambda i: (0, 0)),
            pl.BlockSpec((1, F), lambda i: (0, 0)),
        ],
        out_specs=pl.BlockSpec((BE, F), lambda i: (i, 0)),
        out_shape=jax.ShapeDtypeStruct((E_PAD, F), jnp.float32),
    )(efeat_pad, WeT, be2)


def _tc_layer1(nfeat, s0, s1, d0, d1, WsaT, WsbT, WnaT, WnbT, b12):
    BN = 2000
    grid = (N // BN,)

    def body(nf, a0, a1, e0, e1, wsa, wsb, wna, wnb, b, z1, g1, dinv):
        deg = e0[...][:, :1] + e1[...][:, :1]
        degc = jnp.maximum(deg, 1.0)
        hn = (a0[...] + a1[...]) / degc
        nfb = nf[...]
        z1[...] = (jnp.dot(nfb, wsa[...], preferred_element_type=jnp.float32)
                   + jnp.dot(hn, wsb[...], preferred_element_type=jnp.float32)
                   + b[...])
        g1[...] = (jnp.dot(nfb, wna[...], preferred_element_type=jnp.float32)
                   + jnp.dot(hn, wnb[...], preferred_element_type=jnp.float32))
        dinv[...] = jnp.broadcast_to(1.0 / degc, (BN, DW))

    row = pl.BlockSpec((BN, F), lambda i: (i, 0))
    degs = pl.BlockSpec((BN, DW), lambda i: (i, 0))
    wspec = pl.BlockSpec((F, F), lambda i: (0, 0))
    return pl.pallas_call(
        body,
        grid=grid,
        in_specs=[row, row, row, row, row, wspec, wspec, wspec, wspec,
                  pl.BlockSpec((1, F), lambda i: (0, 0))],
        out_specs=[row, row, degs],
        out_shape=[
            jax.ShapeDtypeStruct((N, F), jnp.float32),
            jax.ShapeDtypeStruct((N, F), jnp.float32),
            jax.ShapeDtypeStruct((N, DW), jnp.float32),
        ],
    )(nfeat, s0, s1, d0, d1, WsaT, WsbT, WnaT, WnbT, b12)


def _tc_layer2(z1, q0, q1, dinv, Ws2T, Wn2T, b22):
    BN = 2000
    grid = (N // BN,)

    def body(z, a0, a1, di, ws, wn, b, z2, g2):
        h1 = jax.nn.relu(z[...] + (a0[...] + a1[...]) * di[...][:, :1])
        z2[...] = (jnp.dot(h1, ws[...], preferred_element_type=jnp.float32)
                   + b[...])
        g2[...] = jnp.dot(h1, wn[...], preferred_element_type=jnp.float32)

    row = pl.BlockSpec((BN, F), lambda i: (i, 0))
    degs = pl.BlockSpec((BN, DW), lambda i: (i, 0))
    wspec = pl.BlockSpec((F, F), lambda i: (0, 0))
    return pl.pallas_call(
        body,
        grid=grid,
        in_specs=[row, row, row, degs, wspec, wspec,
                  pl.BlockSpec((1, F), lambda i: (0, 0))],
        out_specs=[row, row],
        out_shape=[
            jax.ShapeDtypeStruct((N, F), jnp.float32),
            jax.ShapeDtypeStruct((N, F), jnp.float32),
        ],
    )(z1, q0, q1, dinv, Ws2T, Wn2T, b22)


def _tc_final(z2, r0, r1, dinv):
    BN = 2000
    grid = (N // BN,)

    def body(z, a0, a1, di, o):
        o[...] = z[...] + (a0[...] + a1[...]) * di[...][:, :1]

    row = pl.BlockSpec((BN, F), lambda i: (i, 0))
    degs = pl.BlockSpec((BN, DW), lambda i: (i, 0))
    return pl.pallas_call(
        body,
        grid=grid,
        in_specs=[row, row, row, degs],
        out_specs=row,
        out_shape=jax.ShapeDtypeStruct((N, F), jnp.float32),
    )(z2, r0, r1, dinv)


def kernel(nfeat, efeat, We, be, Ws1, Wn1, b1, Ws2, Wn2, b2, edge_index):
    pad = E_PAD - E
    src = jnp.concatenate([edge_index[0], jnp.zeros((pad,), jnp.int32)])
    dst = jnp.concatenate([edge_index[1], jnp.full((pad,), N, jnp.int32)])
    efp = jnp.concatenate([efeat, jnp.zeros((pad, EF), jnp.float32)], axis=0)

    e = _tc_edge_encoder(efp, We.T, be.reshape(1, F))
    degp = _sc_degree(dst)
    sump = _sc_edge_pass_a(nfeat, e, src, dst)
    z1, g1, dinv = _tc_layer1(
        nfeat, sump[0], sump[1], degp[0], degp[1],
        Ws1[:, :F].T, Ws1[:, F:].T, Wn1[:, :F].T, Wn1[:, F:].T,
        b1.reshape(1, F))
    qp = _sc_gather_scatter(g1, src, dst)
    z2, g2 = _tc_layer2(z1, qp[0], qp[1], dinv, Ws2.T, Wn2.T, b2.reshape(1, F))
    rp = _sc_gather_scatter(g2, src, dst)
    return _tc_final(z2, rp[0], rp[1], dinv)


# speedup vs baseline: 1.8572x; 1.8572x over previous
import functools

import jax
import jax.numpy as jnp
from jax import lax
from jax.experimental import pallas as pl
from jax.experimental.pallas import tpu as pltpu
from jax.experimental.pallas import tpu_sc as plsc

N = 10000
E = 320000
F = 128
EF = 16

NC = 2
NS = 16
NW = NC * NS
C = 128
CA = 96
EPW = 10368
E_PAD = NW * EPW
ACC_ROWS = 10112
ZPT = ACC_ROWS // NS
RPT = 624
TAIL = N - NS * RPT
DW = 16


def _zero_vec():
    return jnp.zeros((16,), jnp.float32)


def _sc_degree(dst):
    mesh = plsc.VectorSubcoreMesh(core_axis_name="c", subcore_axis_name="s")

    @functools.partial(
        pl.kernel,
        out_type=jax.ShapeDtypeStruct((NC, N, F), jnp.float32),
        mesh=mesh,
        scratch_types=[
            pltpu.VMEM((2, C), jnp.int32),
            pltpu.VMEM((C, F), jnp.float32),
            pltpu.VMEM_SHARED((ACC_ROWS, F), jnp.float32),
            pltpu.SemaphoreType.DMA,
            pltpu.SemaphoreType.DMA,
        ],
    )
    def kern(dst_hbm, out_deg, dstv, ones, accd, ss0, ss1):
        cid = lax.axis_index("c")
        sid = lax.axis_index("s")
        wid = sid * NC + cid
        ss = (ss0, ss1)
        zero = _zero_vec()

        def zrow(r, carry):
            for j in range(F // 16):
                ones[r, pl.ds(j * 16, 16)] = zero
            return carry
        lax.fori_loop(0, C, zrow, 0)
        for i in range((ZPT + C - 1) // C):
            nr = min(C, ZPT - i * C)
            r0 = sid * ZPT + i * C
            pltpu.sync_copy(ones.at[pl.ds(0, nr)], accd.at[pl.ds(r0, nr)])

        one = jnp.ones((16,), jnp.float32)

        def initrow(r, carry):
            for j in range(F // 16):
                ones[r, pl.ds(j * 16, 16)] = one
            return carry
        lax.fori_loop(0, C, initrow, 0)
        plsc.subcore_barrier()

        base = wid * EPW

        cpw = EPW // C

        def sched(k, p, first):
            if not first:
                pltpu.make_async_copy(ones, accd.at[dstv.at[p]], ss[p]).wait()
            pltpu.sync_copy(dst_hbm.at[pl.ds(base + k * C, C)], dstv.at[p])
            pltpu.async_copy(ones, accd.at[dstv.at[p]], ss[p], add=True)

        sched(0, 0, True)
        sched(1, 1, True)

        def outer(i, carry):
            sched(2 * i + 2, 0, False)
            sched(2 * i + 3, 1, False)
            return carry
        lax.fori_loop(0, (cpw - 3) // 2, outer, 0)
        sched(cpw - 1, 0, False)
        pltpu.make_async_copy(ones, accd.at[dstv.at[0]], ss[0]).wait()
        pltpu.make_async_copy(ones, accd.at[dstv.at[1]], ss[1]).wait()
        plsc.subcore_barrier()

        def cpout(row, nr):
            pltpu.sync_copy(accd.at[pl.ds(row, nr)], ones.at[pl.ds(0, nr)])
            pltpu.sync_copy(ones.at[pl.ds(0, nr)], out_deg.at[cid, pl.ds(row, nr)])

        for i in range((RPT + C - 1) // C):
            cpout(sid * RPT + i * C, min(C, RPT - i * C))

        @pl.when(sid == NS - 1)
        def _tail():
            cpout(NS * RPT, TAIL)

    return kern(dst)


def _sc_edge_pass(table, src, dst, e=None):
    with_mul = e is not None
    cc = CA if with_mul else C
    cpw = EPW // cc
    mesh = plsc.VectorSubcoreMesh(core_axis_name="c", subcore_axis_name="s")

    scratch = [
        pltpu.VMEM((2, cc), jnp.int32),
        pltpu.VMEM((2, cc), jnp.int32),
        pltpu.VMEM((2, cc, F), jnp.float32),
    ]
    if with_mul:
        scratch.append(pltpu.VMEM((2, cc, F), jnp.float32))
    scratch.append(pltpu.VMEM_SHARED((ACC_ROWS, F), jnp.float32))
    nsem = 6 if with_mul else 4
    scratch += [pltpu.SemaphoreType.DMA] * nsem

    @functools.partial(
        pl.kernel,
        out_type=jax.ShapeDtypeStruct((NC, N, F), jnp.float32),
        mesh=mesh,
        scratch_types=scratch,
    )
    def kern(*args):
        if with_mul:
            (table_hbm, e_hbm, src_hbm, dst_hbm, out_sum,
             srcv, dstv, rows, ev, acc, g0, g1, s0, s1, e0, e1) = args
            sg, ss, se = (g0, g1), (s0, s1), (e0, e1)
        else:
            (table_hbm, src_hbm, dst_hbm, out_sum,
             srcv, dstv, rows, acc, g0, g1, s0, s1) = args
            sg, ss = (g0, g1), (s0, s1)
        cid = lax.axis_index("c")
        sid = lax.axis_index("s")
        wid = sid * NC + cid
        zero = _zero_vec()
        base = wid * EPW

        def initrow(r, carry):
            for j in range(F // 16):
                rows[0, r, pl.ds(j * 16, 16)] = zero
            return carry
        lax.fori_loop(0, cc, initrow, 0)

        for i in range((ZPT + cc - 1) // cc):
            nr = min(cc, ZPT - i * cc)
            r0 = sid * ZPT + i * cc
            pltpu.sync_copy(rows.at[0, pl.ds(0, nr)], acc.at[pl.ds(r0, nr)])
        plsc.subcore_barrier()

        def stage_a(k, p, first):
            if not first:
                pltpu.make_async_copy(rows.at[p], acc.at[dstv.at[p]],
                                      ss[p]).wait()
            off = base + k * cc
            pltpu.sync_copy(src_hbm.at[pl.ds(off, cc)], srcv.at[p])
            pltpu.sync_copy(dst_hbm.at[pl.ds(off, cc)], dstv.at[p])
            pltpu.async_copy(table_hbm.at[srcv.at[p]], rows.at[p], sg[p])
            if with_mul:
                pltpu.async_copy(e_hbm.at[pl.ds(off, cc)], ev.at[p], se[p])

        def stage_b(p):
            pltpu.make_async_copy(table_hbm.at[srcv.at[p]], rows.at[p],
                                  sg[p]).wait()
            if with_mul:
                pltpu.make_async_copy(e_hbm.at[pl.ds(base, cc)], ev.at[p],
                                      se[p]).wait()

                def mrow(r, c2):
                    for j in range(F // 16):
                        s = pl.ds(j * 16, 16)
                        rows[p, r, s] = rows[p, r, s] * ev[p, r, s]
                    return c2
                lax.fori_loop(0, cc, mrow, 0, unroll=2)
            pltpu.async_copy(rows.at[p], acc.at[dstv.at[p]], ss[p], add=True)

        stage_a(0, 0, True)
        stage_a(1, 1, True)

        nfull = (cpw - 2) // 2

        def outer(i, carry):
            k = 2 * i
            stage_b(0)
            stage_a(k + 2, 0, False)
            stage_b(1)
            stage_a(k + 3, 1, False)
            return carry
        lax.fori_loop(0, nfull, outer, 0)
        if cpw % 2 == 0:
            stage_b(0)
            stage_b(1)
        else:
            stage_b(0)
            stage_a(cpw - 1, 0, False)
            stage_b(1)
            stage_b(0)
        pltpu.make_async_copy(rows.at[0], acc.at[dstv.at[0]], ss[0]).wait()
        pltpu.make_async_copy(rows.at[1], acc.at[dstv.at[1]], ss[1]).wait()
        plsc.subcore_barrier()

        def cpout(row, nr):
            pltpu.sync_copy(acc.at[pl.ds(row, nr)], rows.at[0, pl.ds(0, nr)])
            pltpu.sync_copy(rows.at[0, pl.ds(0, nr)],
                            out_sum.at[cid, pl.ds(row, nr)])

        for i in range((RPT + cc - 1) // cc):
            cpout(sid * RPT + i * cc, min(cc, RPT - i * cc))

        @pl.when(sid == NS - 1)
        def _tail():
            cpout(NS * RPT, TAIL)

    if with_mul:
        return kern(table, e, src, dst)
    return kern(table, src, dst)


def _sc_edge_pass_a(nfeat, e, src, dst):
    return _sc_edge_pass(nfeat, src, dst, e=e)


def _sc_gather_scatter(table, src, dst):
    return _sc_edge_pass(table, src, dst)


def _tc_edge_encoder(efeat_pad, WeT, be2):
    BE = 2048
    grid = (E_PAD // BE,)

    def body(ef, w, b, o):
        o[...] = jax.nn.relu(
            jnp.dot(ef[...], w[...], preferred_element_type=jnp.float32) + b[...])

    return pl.pallas_call(
        body,
        grid=grid,
        in_specs=[
            pl.BlockSpec((BE, EF), lambda i: (i, 0)),
            pl.BlockSpec((EF, F), lambda i: (0, 0)),
            pl.BlockSpec((1, F), lambda i: (0, 0)),
        ],
        out_specs=pl.BlockSpec((BE, F), lambda i: (i, 0)),
        out_shape=jax.ShapeDtypeStruct((E_PAD, F), jnp.float32),
    )(efeat_pad, WeT, be2)


def _tc_layer1(nfeat, s0, s1, d0, d1, WsaT, WsbT, WnaT, WnbT, b12):
    BN = 2000
    grid = (N // BN,)

    def body(nf, a0, a1, e0, e1, wsa, wsb, wna, wnb, b, z1, g1, dinv):
        deg = e0[...][:, :1] + e1[...][:, :1]
        degc = jnp.maximum(deg, 1.0)
        hn = (a0[...] + a1[...]) / degc
        nfb = nf[...]
        z1[...] = (jnp.dot(nfb, wsa[...], preferred_element_type=jnp.float32)
                   + jnp.dot(hn, wsb[...], preferred_element_type=jnp.float32)
                   + b[...])
        g1[...] = (jnp.dot(nfb, wna[...], preferred_element_type=jnp.float32)
                   + jnp.dot(hn, wnb[...], preferred_element_type=jnp.float32))
        dinv[...] = jnp.broadcast_to(1.0 / degc, (BN, DW))

    row = pl.BlockSpec((BN, F), lambda i: (i, 0))
    degs = pl.BlockSpec((BN, DW), lambda i: (i, 0))
    wspec = pl.BlockSpec((F, F), lambda i: (0, 0))
    return pl.pallas_call(
        body,
        grid=grid,
        in_specs=[row, row, row, row, row, wspec, wspec, wspec, wspec,
                  pl.BlockSpec((1, F), lambda i: (0, 0))],
        out_specs=[row, row, degs],
        out_shape=[
            jax.ShapeDtypeStruct((N, F), jnp.float32),
            jax.ShapeDtypeStruct((N, F), jnp.float32),
            jax.ShapeDtypeStruct((N, DW), jnp.float32),
        ],
    )(nfeat, s0, s1, d0, d1, WsaT, WsbT, WnaT, WnbT, b12)


def _tc_layer2(z1, q0, q1, dinv, Ws2T, Wn2T, b22):
    BN = 2000
    grid = (N // BN,)

    def body(z, a0, a1, di, ws, wn, b, z2, g2):
        h1 = jax.nn.relu(z[...] + (a0[...] + a1[...]) * di[...][:, :1])
        z2[...] = (jnp.dot(h1, ws[...], preferred_element_type=jnp.float32)
                   + b[...])
        g2[...] = jnp.dot(h1, wn[...], preferred_element_type=jnp.float32)

    row = pl.BlockSpec((BN, F), lambda i: (i, 0))
    degs = pl.BlockSpec((BN, DW), lambda i: (i, 0))
    wspec = pl.BlockSpec((F, F), lambda i: (0, 0))
    return pl.pallas_call(
        body,
        grid=grid,
        in_specs=[row, row, row, degs, wspec, wspec,
                  pl.BlockSpec((1, F), lambda i: (0, 0))],
        out_specs=[row, row],
        out_shape=[
            jax.ShapeDtypeStruct((N, F), jnp.float32),
            jax.ShapeDtypeStruct((N, F), jnp.float32),
        ],
    )(z1, q0, q1, dinv, Ws2T, Wn2T, b22)


def _tc_final(z2, r0, r1, dinv):
    BN = 2000
    grid = (N // BN,)

    def body(z, a0, a1, di, o):
        o[...] = z[...] + (a0[...] + a1[...]) * di[...][:, :1]

    row = pl.BlockSpec((BN, F), lambda i: (i, 0))
    degs = pl.BlockSpec((BN, DW), lambda i: (i, 0))
    return pl.pallas_call(
        body,
        grid=grid,
        in_specs=[row, row, row, degs],
        out_specs=row,
        out_shape=jax.ShapeDtypeStruct((N, F), jnp.float32),
    )(z2, r0, r1, dinv)


def kernel(nfeat, efeat, We, be, Ws1, Wn1, b1, Ws2, Wn2, b2, edge_index):
    pad = E_PAD - E
    ar = jnp.arange(pad, dtype=jnp.int32)
    src = jnp.concatenate([edge_index[0], ar % N])
    dst = jnp.concatenate([edge_index[1], N + ar % (ACC_ROWS - N)])
    efp = jnp.concatenate([efeat, jnp.zeros((pad, EF), jnp.float32)], axis=0)

    e = _tc_edge_encoder(efp, We.T, be.reshape(1, F))
    degp = _sc_degree(dst)
    sump = _sc_edge_pass_a(nfeat, e, src, dst)
    z1, g1, dinv = _tc_layer1(
        nfeat, sump[0], sump[1], degp[0], degp[1],
        Ws1[:, :F].T, Ws1[:, F:].T, Wn1[:, :F].T, Wn1[:, F:].T,
        b1.reshape(1, F))
    qp = _sc_gather_scatter(g1, src, dst)
    z2, g2 = _tc_layer2(z1, qp[0], qp[1], dinv, Ws2.T, Wn2.T, b2.reshape(1, F))
    rp = _sc_gather_scatter(g2, src, dst)
    return _tc_final(z2, rp[0], rp[1], dinv)
